# Initial kernel scaffold; baseline (speedup 1.0000x reference)
import jax
import jax.numpy as jnp
from jax.experimental import pallas as pl


def kernel(node_context, original_data, cell_context, high_mask, low_mask, params):
    raise NotImplementedError("write your pallas kernel here")


# speedup vs baseline: 162.5925x; 162.5925x over previous
import numpy as np

import jax
import jax.numpy as jnp
from jax import lax
from jax.experimental import pallas as pl

_B, _NC, _L, _E = 16, 10, 10, 128
_HIGH_STEPS = 9
_C = 10.0

_GUMBEL_TABLES = None


def _gumbel_tables():
    global _GUMBEL_TABLES
    if _GUMBEL_TABLES is not None:
        return _GUMBEL_TABLES

    def build():
        key = jax.random.key(42)
        gh, gl = [], []
        for _ in range(_HIGH_STEPS):
            key, sk = jax.random.split(key)
            gh.append(jax.random.gumbel(sk, (_B, _NC), jnp.float32))
            rows = []
            for _bid in range(_B):
                key, sk2 = jax.random.split(key)
                lk = sk2
                cols = []
                for _s in range(_L):
                    lk, sks = jax.random.split(lk)
                    cols.append(jax.random.gumbel(sks, (1, _L), jnp.float32)[0])
                rows.append(jnp.stack(cols))
            gl.append(jnp.stack(rows))
        return jnp.stack(gh), jnp.stack(gl)

    try:
        cpu = jax.local_devices(backend="cpu")[0]
        with jax.default_device(cpu), jax.ensure_compile_time_eval():
            gh, gl = build()
            gh, gl = np.asarray(gh), np.asarray(gl)
    except Exception:
        with jax.ensure_compile_time_eval():
            gh, gl = build()
            gh, gl = np.asarray(gh), np.asarray(gl)
    _GUMBEL_TABLES = (gh, gl)
    return _GUMBEL_TABLES


def _decoder_kernel(
    nc_ref, ox_ref, oy_ref, cc_ref, hm_ref, lm_ref,
    whc_ref, bhc_ref, wva_ref, wvb_ref, bvw_ref, initw_ref,
    hwq_ref, hbq_ref, hwk_ref, hbk_ref, hv_ref,
    wlh_ref, blh_ref, wlva_ref, wlvb_ref, blv_ref, linitw_ref,
    lwq_ref, lbq_ref, lwk_ref, lbk_ref, lv_ref,
    gh_ref, gl_ref,
    out_ref,
):
    f32 = jnp.float32
    iota = lax.broadcasted_iota(jnp.int32, (_B, _NC), 1)

    def matmul(a, w):
        return jnp.dot(a, w, preferred_element_type=f32)

    def argmax_onehot(z):
        m = jnp.max(z, axis=-1, keepdims=True)
        cand = jnp.where(z >= m, iota, 10000)
        idx = jnp.min(cand, axis=-1, keepdims=True)
        return (iota == idx).astype(f32)

    def pointer_probs(query, k, wq, bq, v, mask):
        q = matmul(query, wq) + bq
        e = jnp.tanh(q[:, None, :] + k)
        u = _C * jnp.tanh(jnp.sum(e * v[None], axis=-1))
        u = u - 1e8 * mask
        um = jnp.max(u, axis=-1, keepdims=True)
        ex = jnp.exp(u - um)
        return ex / jnp.sum(ex, axis=-1, keepdims=True)

    cc = cc_ref[...]
    hv = hv_ref[...]
    lv = lv_ref[...]
    wva, wvb, bvw = wva_ref[...], wvb_ref[...], bvw_ref[...]
    wlva, wlvb, blv = wlva_ref[...], wlvb_ref[...], blv_ref[...]
    hwq, hbq = hwq_ref[...], hbq_ref[...]
    lwq, lbq = lwq_ref[...], lbq_ref[...]
    lwk, lbk = lwk_ref[...], lbk_ref[...]

    nc_cells = [nc_ref[:, c] for c in range(_NC)]
    ox_cells = [ox_ref[:, c] for c in range(_NC)]
    oy_cells = [oy_ref[:, c] for c in range(_NC)]
    lm_cells = [lm_ref[:, c] for c in range(_NC)]

    h_mean = jnp.mean(cc, axis=1)
    h_bar = matmul(h_mean, whc_ref[...]) + bhc_ref[...]
    initw = initw_ref[...]
    h_rest0 = matmul(initw[0:1], wva) + matmul(initw[1:2], wvb) + bvw
    query = h_bar + h_rest0
    hmask = jnp.where(iota == 0, 1.0, hm_ref[...])
    k_high = jnp.reshape(
        matmul(jnp.reshape(cc, (_B * _NC, _E)), hwk_ref[...]),
        (_B, _NC, _E)) + hbk_ref[...][None]

    linitw = linitw_ref[...]
    lh_rest0 = matmul(linitw[0:1], wlva) + matmul(linitw[1:2], wlvb) + blv

    lx = jnp.zeros((_B, 1), f32)
    ly = jnp.zeros((_B, 1), f32)
    cell_lp = f32(0.0)
    init_h = None
    total_lp = f32(0.0)
    total_reward = f32(0.0)

    for i in range(_HIGH_STEPS):
        prob = pointer_probs(query, k_high, hwq, hbq, hv, hmask)
        logits = jnp.log(prob + 1e-10)
        oh = argmax_onehot(logits + gh_ref[i])
        cell_lp = cell_lp + jnp.sum(oh * logits)

        cur_cell = oh[:, 0:1, None] * nc_cells[0]
        ocx = oh[:, 0:1] * ox_cells[0]
        ocy = oh[:, 0:1] * oy_cells[0]
        lmask = oh[:, 0:1] * lm_cells[0]
        for c in range(1, _NC):
            w = oh[:, c:c + 1]
            cur_cell = cur_cell + w[:, :, None] * nc_cells[c]
            ocx = ocx + w * ox_cells[c]
            ocy = ocy + w * oy_cells[c]
            lmask = lmask + w * lm_cells[c]

        h_mean_l = jnp.mean(cur_cell, axis=1)
        h_bar_l = matmul(h_mean_l, wlh_ref[...]) + blh_ref[...]
        query_l = h_bar_l + lh_rest0
        k_low = jnp.reshape(
            matmul(jnp.reshape(cur_cell, (_B * _L, _E)), lwk),
            (_B, _L, _E)) + lbk[None]

        oh_cur = (iota == 0).astype(f32)
        local_r = jnp.zeros((_B, 1), f32)
        llp = f32(0.0)
        init_h_l = None
        ix = iy = fx = fy = None
        gl_i = gl_ref[i]
        for s in range(_L):
            probl = pointer_probs(query_l, k_low, lwq, lbq, lv, lmask)
            logitsl = jnp.log(probl + 1e-10)
            ohs = argmax_onehot(logitsl + gl_i[:, s])
            llp = llp + jnp.sum(ohs * logitsl)
            lmask = lmask * (1.0 - ohs) + ohs
            h = jnp.sum(ohs[:, :, None] * cur_cell, axis=1)
            if s == 0:
                init_h_l = h
            query_l = h_bar_l + matmul(init_h_l, wlva) + matmul(h, wlvb) + blv
            cx = jnp.sum(oh_cur * ocx, axis=-1, keepdims=True)
            cy = jnp.sum(oh_cur * ocy, axis=-1, keepdims=True)
            nx = jnp.sum(ohs * ocx, axis=-1, keepdims=True)
            ny = jnp.sum(ohs * ocy, axis=-1, keepdims=True)
            local_r = local_r + jnp.sqrt(
                (nx - cx) ** 2 + (ny - cy) ** 2 + 1e-12)
            if s == 0:
                ix, iy = cx, cy
            if s == _L - 1:
                fx, fy = nx, ny
            oh_cur = ohs

        cell_reward = jnp.sum(jnp.sqrt((lx - ix) ** 2 + (ly - iy) ** 2 + 1e-12))
        lx, ly = fx, fy

        hmask = hmask * (1.0 - oh) + oh
        h_hi = jnp.sum(oh[:, :, None] * cc, axis=1)
        if i == 0:
            init_h = h_hi
        query = h_bar + matmul(init_h, wva) + matmul(h_hi, wvb) + bvw
        total_reward = cell_reward + jnp.sum(local_r)
        total_lp = cell_lp + llp

    out_iota = lax.broadcasted_iota(jnp.int32, (1, _E), 1)
    out_ref[...] = jnp.where(
        out_iota == 0, total_lp,
        jnp.where(out_iota == 1, total_reward, 0.0))


def kernel(node_context, original_data, cell_context, high_mask, low_mask, params):
    gh_np, gl_np = _gumbel_tables()
    f32 = jnp.float32

    def r2(v):
        return jnp.reshape(v, (1, -1)).astype(f32)

    hp, lp = params['high_ptr'], params['low_ptr']
    w_vw, b_vw = params['v_w']
    w_lvw, b_lvw = params['low_v_w']

    args = (
        node_context.astype(f32),
        original_data[..., 0].astype(f32),
        original_data[..., 1].astype(f32),
        cell_context.astype(f32),
        high_mask.astype(f32),
        low_mask.astype(f32),
        params['h_ctx'][0].astype(f32), r2(params['h_ctx'][1]),
        w_vw[:_E].astype(f32), w_vw[_E:].astype(f32), r2(b_vw),
        jnp.reshape(params['init_w'], (2, _E)).astype(f32),
        hp['Wq'].astype(f32), r2(hp['bq']), hp['Wk'].astype(f32),
        r2(hp['bk']), r2(hp['V']),
        params['low_h_ctx'][0].astype(f32), r2(params['low_h_ctx'][1]),
        w_lvw[:_E].astype(f32), w_lvw[_E:].astype(f32), r2(b_lvw),
        jnp.reshape(params['low_init_w'], (2, _E)).astype(f32),
        lp['Wq'].astype(f32), r2(lp['bq']), lp['Wk'].astype(f32),
        r2(lp['bk']), r2(lp['V']),
        jnp.asarray(gh_np), jnp.asarray(gl_np),
    )

    out = pl.pallas_call(
        _decoder_kernel,
        out_shape=jax.ShapeDtypeStruct((1, _E), f32),
    )(*args)

    total_log_prob = out[0, 0:1]
    total_reward = out[0, 1:2]
    return total_log_prob, total_reward


# speedup vs baseline: 228.7475x; 1.4069x over previous
import numpy as np

import jax
import jax.numpy as jnp
from jax import lax
from jax.experimental import pallas as pl

_B, _NC, _L, _E = 16, 10, 10, 128
_HIGH_STEPS = 9
_C = 10.0


_ROTS = ((13, 15, 26, 6), (17, 29, 16, 24))


def _threefry2x32(k0, k1, x0, x1):
    x0 = x0.astype(np.uint32).copy()
    x1 = x1.astype(np.uint32).copy()
    ks = (np.uint32(k0), np.uint32(k1),
          np.uint32(k0) ^ np.uint32(k1) ^ np.uint32(0x1BD11BDA))
    x0 += ks[0]
    x1 += ks[1]
    for d in range(5):
        for r in _ROTS[d % 2]:
            x0 += x1
            x1 = ((x1 << np.uint32(r)) | (x1 >> np.uint32(32 - r))).astype(np.uint32)
            x1 ^= x0
        x0 += ks[(d + 1) % 3]
        x1 += ks[(d + 2) % 3] + np.uint32(d + 1)
    return x0, x1


def _np_split(key):
    a, b = _threefry2x32(key[0], key[1],
                         np.zeros(2, np.uint32), np.arange(2, dtype=np.uint32))
    return (a[0], b[0]), (a[1], b[1])


def _np_gumbel(key, shape):
    size = int(np.prod(shape))
    j = np.arange(size, dtype=np.uint64)
    hi = (j >> np.uint64(32)).astype(np.uint32)
    lo = (j & np.uint64(0xFFFFFFFF)).astype(np.uint32)
    a, b = _threefry2x32(key[0], key[1], hi, lo)
    bits = a ^ b
    f = ((bits >> np.uint32(9)) | np.uint32(0x3F800000)).view(np.float32) - np.float32(1.0)
    tiny = np.float32(np.finfo(np.float32).tiny)
    u = np.maximum(tiny, f * (np.float32(1.0) - tiny) + tiny)
    return (-np.log(-np.log(u))).astype(np.float32).reshape(shape)


_GUMBEL_TABLES = None


def _gumbel_tables():
    global _GUMBEL_TABLES
    if _GUMBEL_TABLES is not None:
        return _GUMBEL_TABLES
    key = (np.uint32(0), np.uint32(42))
    gh = np.zeros((_HIGH_STEPS, _B, _NC), np.float32)
    gl = np.zeros((_HIGH_STEPS, _B, _L, _NC), np.float32)
    for i in range(_HIGH_STEPS):
        key, sk = _np_split(key)
        gh[i] = _np_gumbel(sk, (_B, _NC))
        for bid in range(_B):
            key, sk2 = _np_split(key)
            lk = sk2
            for s in range(_L):
                lk, sks = _np_split(lk)
                gl[i, bid, s] = _np_gumbel(sks, (1, _NC))[0]
    _GUMBEL_TABLES = (gh, gl)
    return _GUMBEL_TABLES


def _decoder_kernel(
    nc_ref, ox_ref, oy_ref, cc_ref, hm_ref, lm_ref,
    whc_ref, bhc_ref, wva_ref, wvb_ref, bvw_ref, initw_ref,
    hwq_ref, hbq_ref, hwk_ref, hbk_ref, hv_ref,
    wlh_ref, blh_ref, wlva_ref, wlvb_ref, blv_ref, linitw_ref,
    lwq_ref, lbq_ref, lwk_ref, lbk_ref, lv_ref,
    gh_ref, gl_ref,
    out_ref,
):
    f32 = jnp.float32
    iota = lax.broadcasted_iota(jnp.int32, (_B, _NC), 1)

    def matmul(a, w):
        return jnp.dot(a, w, preferred_element_type=f32)

    def argmax_onehot(z):
        m = jnp.max(z, axis=-1, keepdims=True)
        cand = jnp.where(z >= m, iota, 10000)
        idx = jnp.min(cand, axis=-1, keepdims=True)
        return (iota == idx).astype(f32)

    def probs_from_q(q, k, v, mask):
        e = jnp.tanh(q[:, None, :] + k)
        u = _C * jnp.tanh(jnp.sum(e * v[None], axis=-1))
        u = u - 1e8 * mask
        um = jnp.max(u, axis=-1, keepdims=True)
        ex = jnp.exp(u - um)
        return ex / jnp.sum(ex, axis=-1, keepdims=True)

    def row_gather(oh, table):
        acc = None
        for c in range(table.shape[1]):
            w = oh[:, c:c + 1]
            sl = table[:, c]
            if sl.ndim == 3:
                w = w[:, :, None]
            term = w * sl
            acc = term if acc is None else acc + term
        return acc

    cc = cc_ref[...]
    hv, lv = hv_ref[...], lv_ref[...]
    wva, wvb, bvw = wva_ref[...], wvb_ref[...], bvw_ref[...]
    wlva, wlvb, blv = wlva_ref[...], wlvb_ref[...], blv_ref[...]
    hwq, hbq = hwq_ref[...], hbq_ref[...]
    lwq, lbq = lwq_ref[...], lbq_ref[...]
    lwk, lbk = lwk_ref[...], lbk_ref[...]

    ox = ox_ref[...]
    oy = oy_ref[...]
    lm = lm_ref[...]

    h_mean = jnp.mean(cc, axis=1)
    h_bar = matmul(h_mean, whc_ref[...]) + bhc_ref[...]
    initw = initw_ref[...]
    h_rest0 = matmul(initw[0:1], wva) + matmul(initw[1:2], wvb) + bvw
    hmask = jnp.where(iota == 0, 1.0, hm_ref[...])

    cc2d = jnp.reshape(cc, (_B * _NC, _E))
    k_high = jnp.reshape(matmul(cc2d, hwk_ref[...]),
                         (_B, _NC, _E)) + hbk_ref[...][None]

    mha = matmul(wva, hwq)
    mhb = matmul(wvb, hwq)
    cc_ma = jnp.reshape(matmul(cc2d, mha), (_B, _NC, _E))
    cc_mb = jnp.reshape(matmul(cc2d, mhb), (_B, _NC, _E))
    q_hi = matmul(h_bar + h_rest0, hwq) + hbq
    base_hi = matmul(h_bar + bvw, hwq) + hbq

    mla = matmul(wlva, lwq)
    mlb = matmul(wlvb, lwq)
    linitw = linitw_ref[...]
    lh_rest0 = matmul(linitw[0:1], wlva) + matmul(linitw[1:2], wlvb) + blv

    lx = jnp.zeros((_B, 1), f32)
    ly = jnp.zeros((_B, 1), f32)
    cell_lp = f32(0.0)
    total_lp = f32(0.0)
    total_reward = f32(0.0)
    base_hi2 = None

    for i in range(_HIGH_STEPS):
        prob = probs_from_q(q_hi, k_high, hv, hmask)
        logits = jnp.log(prob + 1e-10)
        oh = argmax_onehot(logits + gh_ref[i])
        cell_lp = cell_lp + jnp.sum(oh * logits)
        hmask = hmask * (1.0 - oh) + oh
        if i == 0:
            base_hi2 = base_hi + row_gather(oh, cc_ma)
        q_hi = base_hi2 + row_gather(oh, cc_mb)

        cur_cell = row_gather(oh, nc_ref[...])
        ocx = row_gather(oh, ox)
        ocy = row_gather(oh, oy)
        lmask = row_gather(oh, lm)

        h_mean_l = jnp.mean(cur_cell, axis=1)
        h_bar_l = matmul(h_mean_l, wlh_ref[...]) + blh_ref[...]
        cur2d = jnp.reshape(cur_cell, (_B * _L, _E))
        k_low = jnp.reshape(matmul(cur2d, lwk), (_B, _L, _E)) + lbk[None]
        cc_mla = jnp.reshape(matmul(cur2d, mla), (_B, _L, _E))
        cc_mlb = jnp.reshape(matmul(cur2d, mlb), (_B, _L, _E))
        q_lo = matmul(h_bar_l + lh_rest0, lwq) + lbq
        base_lo = matmul(h_bar_l + blv, lwq) + lbq

        oh_cur = (iota == 0).astype(f32)
        local_r = jnp.zeros((_B, 1), f32)
        llp = f32(0.0)
        base_lo2 = None
        ix = iy = fx = fy = None
        for s in range(_L):
            probl = probs_from_q(q_lo, k_low, lv, lmask)
            logitsl = jnp.log(probl + 1e-10)
            ohs = argmax_onehot(logitsl + gl_ref[i, :, s])
            llp = llp + jnp.sum(ohs * logitsl)
            lmask = lmask * (1.0 - ohs) + ohs
            if s == 0:
                base_lo2 = base_lo + row_gather(ohs, cc_mla)
            q_lo = base_lo2 + row_gather(ohs, cc_mlb)
            cx = jnp.sum(oh_cur * ocx, axis=-1, keepdims=True)
            cy = jnp.sum(oh_cur * ocy, axis=-1, keepdims=True)
            nx = jnp.sum(ohs * ocx, axis=-1, keepdims=True)
            ny = jnp.sum(ohs * ocy, axis=-1, keepdims=True)
            local_r = local_r + jnp.sqrt(
                (nx - cx) ** 2 + (ny - cy) ** 2 + 1e-12)
            if s == 0:
                ix, iy = cx, cy
            if s == _L - 1:
                fx, fy = nx, ny
            oh_cur = ohs

        cell_reward = jnp.sum(jnp.sqrt((lx - ix) ** 2 + (ly - iy) ** 2 + 1e-12))
        lx, ly = fx, fy
        total_reward = cell_reward + jnp.sum(local_r)
        total_lp = cell_lp + llp

    out_iota = lax.broadcasted_iota(jnp.int32, (1, _E), 1)
    out_ref[...] = jnp.where(
        out_iota == 0, total_lp,
        jnp.where(out_iota == 1, total_reward, 0.0))


def kernel(node_context, original_data, cell_context, high_mask, low_mask, params):
    gh_np, gl_np = _gumbel_tables()
    f32 = jnp.float32

    def r2(v):
        return jnp.reshape(v, (1, -1)).astype(f32)

    hp, lp = params['high_ptr'], params['low_ptr']
    w_vw, b_vw = params['v_w']
    w_lvw, b_lvw = params['low_v_w']

    args = (
        node_context.astype(f32),
        original_data[..., 0].astype(f32),
        original_data[..., 1].astype(f32),
        cell_context.astype(f32),
        high_mask.astype(f32),
        low_mask.astype(f32),
        params['h_ctx'][0].astype(f32), r2(params['h_ctx'][1]),
        w_vw[:_E].astype(f32), w_vw[_E:].astype(f32), r2(b_vw),
        jnp.reshape(params['init_w'], (2, _E)).astype(f32),
        hp['Wq'].astype(f32), r2(hp['bq']), hp['Wk'].astype(f32),
        r2(hp['bk']), r2(hp['V']),
        params['low_h_ctx'][0].astype(f32), r2(params['low_h_ctx'][1]),
        w_lvw[:_E].astype(f32), w_lvw[_E:].astype(f32), r2(b_lvw),
        jnp.reshape(params['low_init_w'], (2, _E)).astype(f32),
        lp['Wq'].astype(f32), r2(lp['bq']), lp['Wk'].astype(f32),
        r2(lp['bk']), r2(lp['V']),
        jnp.asarray(gh_np), jnp.asarray(gl_np),
    )

    out = pl.pallas_call(
        _decoder_kernel,
        out_shape=jax.ShapeDtypeStruct((1, _E), f32),
    )(*args)

    total_log_prob = out[0, 0:1]
    total_reward = out[0, 1:2]
    return total_log_prob, total_reward
